# baseline (speedup 1.0000x reference)
import functools

import jax
import jax.numpy as jnp
from jax import lax
from jax.experimental import pallas as pl
from jax.experimental.pallas import tpu as pltpu
from jax.experimental.pallas import tpu_sc as plsc

_N = 10000
_E = 320000
_DIN = 128
_DE = 16
_D = 256
_B = 16
_H = 128
_OUT = 1
_L = 3
_NPAD = 10240

_F32 = jnp.float32
_I32 = jnp.int32

_SIZES = [10080] * 5 + [9968] * 10 + [9920]
_STARTS = [0]
for _s in _SIZES:
    _STARTS.append(_STARTS[-1] + _s)
_HALF_E = _E // 2


def _edge_body(ea_ref, we1_ref, we2_ref, e_ref):
    e_ref[...] = jax.nn.silu(ea_ref[...] @ we1_ref[...]) @ we2_ref[...]


def _tc_edge(edge_attr, We1, We2):
    te = 2560
    return pl.pallas_call(
        _edge_body,
        grid=(_E // te,),
        in_specs=[
            pl.BlockSpec((te, _DE), lambda i: (i, 0)),
            pl.BlockSpec((_DE, _D), lambda i: (0, 0)),
            pl.BlockSpec((_D, _D), lambda i: (0, 0)),
        ],
        out_specs=pl.BlockSpec((te, _D), lambda i: (i, 0)),
        out_shape=jax.ShapeDtypeStruct((_E, _D), _F32),
    )(edge_attr, We1, We2)


def _h0_body(x_ref, wenc_ref, benc_ref, h_ref):
    h_ref[...] = x_ref[...] @ wenc_ref[...] + benc_ref[...]


def _tc_h0(x, Wenc, benc2):
    tn = 1000
    return pl.pallas_call(
        _h0_body,
        grid=(_N // tn,),
        in_specs=[
            pl.BlockSpec((tn, _DIN), lambda i: (i, 0)),
            pl.BlockSpec((_DIN, _D), lambda i: (0, 0)),
            pl.BlockSpec((1, _D), lambda i: (0, 0)),
        ],
        out_specs=pl.BlockSpec((tn, _D), lambda i: (i, 0)),
        out_shape=jax.ShapeDtypeStruct((_N, _D), _F32),
    )(x, Wenc, benc2)


def _msg_body(ge_ref, wmsg_ref, m_ref):
    m_ref[...] = jax.nn.silu(ge_ref[...] @ wmsg_ref[...])


def _tc_msg(ge, Wmsg_l):
    te = 2560
    return pl.pallas_call(
        _msg_body,
        grid=(_E // te,),
        in_specs=[
            pl.BlockSpec((te, _D), lambda i: (i, 0)),
            pl.BlockSpec((_D, _D), lambda i: (0, 0)),
        ],
        out_specs=pl.BlockSpec((te, _D), lambda i: (i, 0)),
        out_shape=jax.ShapeDtypeStruct((_E, _D), _F32),
    )(ge, Wmsg_l)


_TN_UPD = 1000


def _upd_body(h_ref, agg_ref, pf_ref, plt_ref, fn_ref, ln_ref, wupd_ref,
              hn_ref):
    i = pl.program_id(0)
    rows = lax.broadcasted_iota(_I32, (_TN_UPD, 1), 0) + i * _TN_UPD
    fn = fn_ref[...]
    ln = ln_ref[...]
    agg = agg_ref[...]
    keep = jnp.ones((_TN_UPD, 1), _F32)
    for c in range(32):
        keep = keep * jnp.where(rows == fn[c * 8, 0], 0.0, 1.0)
        keep = keep * jnp.where(rows == ln[c * 8, 0], 0.0, 1.0)
    agg = agg * keep
    for c in range(32):
        agg = agg + jnp.where(rows == fn[c * 8, 0], 1.0, 0.0) * pf_ref[c * 8]
        agg = agg + jnp.where(rows == ln[c * 8, 0], 1.0, 0.0) * plt_ref[c * 8]
    hn_ref[...] = h_ref[...] + jax.nn.silu(agg @ wupd_ref[...])


def _tc_update(h, agg_raw, partsF, partsL, fnid, lnid, Wupd_l):
    return pl.pallas_call(
        _upd_body,
        grid=(_N // _TN_UPD,),
        in_specs=[
            pl.BlockSpec((_TN_UPD, _D), lambda i: (i, 0)),
            pl.BlockSpec((_TN_UPD, _D), lambda i: (i, 0)),
            pl.BlockSpec((256, _D), lambda i: (0, 0)),
            pl.BlockSpec((256, _D), lambda i: (0, 0)),
            pl.BlockSpec((256, 16), lambda i: (0, 0)),
            pl.BlockSpec((256, 16), lambda i: (0, 0)),
            pl.BlockSpec((_D, _D), lambda i: (0, 0)),
        ],
        out_specs=pl.BlockSpec((_TN_UPD, _D), lambda i: (i, 0)),
        out_shape=jax.ShapeDtypeStruct((_N, _D), _F32),
    )(h, agg_raw, partsF, partsL, fnid, lnid, Wupd_l)


def _pool_body(h_ref, gid_ref, w1_ref, b1_ref, w2_ref, b2_ref, w3_ref, b3_ref,
               out_ref):
    ids = gid_ref[...]
    iota = lax.broadcasted_iota(_I32, (_N, _B), 1)
    onehot = (ids == iota).astype(_F32)
    pooled = lax.dot_general(onehot, h_ref[...], (((0,), (0,)), ((), ())),
                             precision=lax.Precision.HIGHEST)
    counts = jnp.maximum(jnp.sum(onehot, axis=0), 1.0)
    pooled = pooled / counts[:, None]
    o = jax.nn.silu(pooled @ w1_ref[...] + b1_ref[...])
    o = jax.nn.silu(o @ w2_ref[...] + b2_ref[...])
    out_ref[...] = o @ w3_ref[...] + b3_ref[...]


def _tc_pool_head(h, gid2, W1, b1_2, W2, b2_2, W3, b3_2):
    return pl.pallas_call(
        _pool_body,
        out_shape=jax.ShapeDtypeStruct((_B, _OUT), _F32),
    )(h, gid2, W1, b1_2, W2, b2_2, W3, b3_2)


_GPT = _E // 32
_GCH = 128
_GFULL = _GPT // _GCH
_GTAIL = _GPT - _GFULL * _GCH


def _sc_gather(h, e, src_s, perm):
    mesh = plsc.VectorSubcoreMesh(core_axis_name="c", subcore_axis_name="s")

    @functools.partial(
        pl.kernel,
        mesh=mesh,
        out_type=jax.ShapeDtypeStruct((_E, _D), _F32),
        scratch_types=[
            pltpu.VMEM((_GCH,), _I32),
            pltpu.VMEM((_GCH,), _I32),
            pltpu.VMEM((_GCH, _D), _F32),
            pltpu.VMEM((_GCH, _D), _F32),
            pltpu.SemaphoreType.DMA,
            pltpu.SemaphoreType.DMA,
        ],
    )
    def k(h_h, e_h, src_h, perm_h, ge_h, sidx, pidx, hbuf, ebuf, sem1, sem2):
        cc = lax.axis_index("c")
        ss = lax.axis_index("s")
        wid = cc * 16 + ss
        base = wid * _GPT

        def do_chunk(e0, cnt):
            pltpu.sync_copy(src_h.at[pl.ds(e0, cnt)], sidx.at[pl.ds(0, cnt)])
            pltpu.sync_copy(perm_h.at[pl.ds(e0, cnt)], pidx.at[pl.ds(0, cnt)])
            cp1 = pltpu.async_copy(h_h.at[sidx.at[pl.ds(0, cnt)]],
                                   hbuf.at[pl.ds(0, cnt)], sem1)
            cp2 = pltpu.async_copy(e_h.at[pidx.at[pl.ds(0, cnt)]],
                                   ebuf.at[pl.ds(0, cnt)], sem2)
            cp1.wait()
            cp2.wait()

            def row(r, carry):
                for j in range(_D // 16):
                    sl = pl.ds(j * 16, 16)
                    hbuf[r, sl] = hbuf[r, sl] + ebuf[r, sl]
                return carry

            lax.fori_loop(0, cnt, row, 0)
            pltpu.sync_copy(hbuf.at[pl.ds(0, cnt)],
                            ge_h.at[pl.ds(e0, cnt)])

        def chunk(i, carry):
            do_chunk(base + i * _GCH, _GCH)
            return carry

        lax.fori_loop(0, _GFULL, chunk, 0)
        do_chunk(base + _GFULL * _GCH, _GTAIL)

    return k(h, e, src_s, perm)


_PART = 80
_SB = 16


def _sc_scatter(m_s, dst_s, nextfn):
    mesh = plsc.VectorSubcoreMesh(core_axis_name="c", subcore_axis_name="s")

    @functools.partial(
        pl.kernel,
        mesh=mesh,
        out_type=[
            jax.ShapeDtypeStruct((_NPAD, _D), _F32),
            jax.ShapeDtypeStruct((256, _D), _F32),
            jax.ShapeDtypeStruct((256, _D), _F32),
            jax.ShapeDtypeStruct((256, 16), _I32),
            jax.ShapeDtypeStruct((256, 16), _I32),
        ],
        scratch_types=[
            pltpu.VMEM((_PART, _D), _F32),
            pltpu.VMEM((_PART + 16,), _I32),
            pltpu.VMEM((8, _D), _F32),
            pltpu.VMEM((_SB, _D), _F32),
            pltpu.VMEM((_SB,), _I32),
            pltpu.VMEM((8, 16), _I32),
            pltpu.VMEM((16,), _I32),
        ],
    )
    def k(m_h, dst_h, nfn_h, agg_h, pf_h, pl_h, fn_h, ln_h,
          mbuf, dbuf, acc, sbuf, sidx, idbuf, nfb):
        cc = lax.axis_index("c")
        ss = lax.axis_index("s")
        chunkid = cc * 16 + ss
        prow = chunkid * 8
        st = jnp.where(ss < 5, 10080 * ss,
                       jnp.where(ss < 15, 50400 + 9968 * (ss - 5), 150080))
        st = st + cc * _HALF_E
        nfull = jnp.where(ss < 5, 126, 124)
        has_tail = jnp.logical_and(ss >= 5, ss < 15)
        dummy = _N + chunkid
        lanes = lax.iota(_I32, 16)
        zv = jnp.zeros((16,), _F32)

        def stage_acc(node, kst):
            for j in range(_D // 16):
                sl = pl.ds(j * 16, 16)
                sbuf[kst, sl] = acc[0, sl]
            sidx[...] = jnp.where(lanes == kst, node, sidx[...])
            return kst + 1

        def stage_zero(node, kst):
            for j in range(_D // 16):
                sbuf[kst, pl.ds(j * 16, 16)] = zv
            sidx[...] = jnp.where(lanes == kst, node, sidx[...])
            return kst + 1

        def maybe_flush(kst):
            @pl.when(kst == _SB)
            def _():
                pltpu.sync_copy(sbuf, agg_h.at[sidx])
            return jnp.where(kst == _SB, 0, kst)

        def gap_zeros(lo, hi, kst):
            def gap(g, kk):
                return maybe_flush(stage_zero(g, kk))
            return lax.fori_loop(lo, hi, gap, kst)

        def write_partial(dst_ref, id_ref, node):
            pltpu.sync_copy(acc, dst_ref.at[pl.ds(prow, 8)])
            idbuf[0, pl.ds(0, 16)] = jnp.full((16,), node, _I32)
            pltpu.sync_copy(idbuf, id_ref.at[pl.ds(prow, 8)])

        def process_part(e0, cnt, carry):
            cur, first_done, kst = carry
            pltpu.sync_copy(m_h.at[pl.ds(e0, cnt)], mbuf.at[pl.ds(0, cnt)])
            pltpu.sync_copy(dst_h.at[pl.ds(e0, cnt)], dbuf.at[pl.ds(0, cnt)])

            def edge(i, carry2):
                cur, first_done, kst = carry2
                d = dbuf[pl.ds(i, 16)][0]
                is_new = d != cur
                do_flush = jnp.logical_and(is_new, cur >= 0)

                def flush_and_gap():
                    def to_partsF():
                        write_partial(pf_h, fn_h, cur)
                        return kst

                    def to_stage():
                        return maybe_flush(stage_acc(cur, kst))

                    k2 = lax.cond(first_done == 0, to_partsF, to_stage)
                    return gap_zeros(cur + 1, d, k2)

                kst = lax.cond(do_flush, flush_and_gap, lambda: kst)

                @pl.when(is_new)
                def _():
                    for j in range(_D // 16):
                        sl = pl.ds(j * 16, 16)
                        acc[0, sl] = mbuf[i, sl]

                @pl.when(jnp.logical_not(is_new))
                def _():
                    for j in range(_D // 16):
                        sl = pl.ds(j * 16, 16)
                        acc[0, sl] = acc[0, sl] + mbuf[i, sl]

                fd = jnp.where(do_flush, 1, first_done)
                cur = jnp.where(is_new, d, cur)
                return (cur, fd, kst)

            return lax.fori_loop(0, cnt, edge, (cur, first_done, kst))

        pltpu.sync_copy(nfn_h.at[chunkid], nfb)
        nextfn = nfb[...][0]

        sidx[...] = jnp.full((16,), dummy, _I32)
        k0 = jnp.int32(0)

        @pl.when(chunkid == 0)
        def _():
            pltpu.sync_copy(dst_h.at[pl.ds(0, 16)], nfb)

        first0 = jnp.where(chunkid == 0, nfb[...][0], 0)

        @pl.when(chunkid == 0)
        def _():
            pltpu.sync_copy(nfn_h.at[chunkid], nfb)

        k0 = gap_zeros(0, first0, k0)

        def part(p, carry):
            return process_part(st + p * _PART, _PART, carry)

        carry = (jnp.int32(-1), jnp.int32(0), k0)
        carry = lax.fori_loop(0, nfull, part, carry)
        cur, first_done, kst = lax.cond(
            has_tail,
            lambda: process_part(st + nfull * _PART, 48, carry),
            lambda: carry)

        @pl.when(first_done == 0)
        def _():
            write_partial(pf_h, fn_h, cur)
            for j in range(_D // 16):
                acc[0, pl.ds(j * 16, 16)] = zv
            write_partial(pl_h, ln_h, -1)

        @pl.when(first_done != 0)
        def _():
            write_partial(pl_h, ln_h, cur)

        kst = gap_zeros(cur + 1, nextfn, kst)

        sidx[...] = jnp.where(lanes >= kst, dummy, sidx[...])
        pltpu.sync_copy(sbuf, agg_h.at[sidx])

    return k(m_s, dst_s, nextfn)


def kernel(x, edge_index, edge_attr, graph_ids, We1, We2, Wenc, benc,
           Wmsg, Wupd, W1, b1, W2, b2, W3, b3):
    src = edge_index[0]
    dst = edge_index[1]
    perm = jnp.argsort(dst, stable=True).astype(_I32)
    src_s = src[perm]
    dst_s = dst[perm]
    chunk_starts = jnp.array(
        [_STARTS[i] for i in range(1, 16)] + [_HALF_E]
        + [_HALF_E + _STARTS[i] for i in range(1, 16)], _I32)
    nextfn = jnp.concatenate([dst_s[chunk_starts], jnp.array([_N], _I32)])
    nextfn = jnp.broadcast_to(nextfn[:, None], (32, 16)).astype(_I32)

    benc2 = benc.reshape(1, _D)
    gid2 = graph_ids.reshape(_N, 1)
    b1_2 = b1.reshape(1, _H)
    b2_2 = b2.reshape(1, _H)
    b3_2 = b3.reshape(1, _OUT)

    e = _tc_edge(edge_attr, We1, We2)
    h = _tc_h0(x, Wenc, benc2)

    for l in range(_L):
        ge = _sc_gather(h, e, src_s, perm)
        m_s = _tc_msg(ge, Wmsg[l])
        agg_raw, partsF, partsL, fnid, lnid = _sc_scatter(m_s, dst_s, nextfn)
        h = _tc_update(h, agg_raw[:_N], partsF, partsL, fnid, lnid, Wupd[l])

    return _tc_pool_head(h, gid2, W1, b1_2, W2, b2_2, W3, b3_2)


# speedup vs baseline: 1.0734x; 1.0734x over previous
import functools

import jax
import jax.numpy as jnp
from jax import lax
from jax.experimental import pallas as pl
from jax.experimental.pallas import tpu as pltpu
from jax.experimental.pallas import tpu_sc as plsc

_N = 10000
_E = 320000
_DIN = 128
_DE = 16
_D = 256
_B = 16
_H = 128
_OUT = 1
_L = 3
_NPAD = 10240

_F32 = jnp.float32
_I32 = jnp.int32

_SIZES = [10080] * 5 + [9968] * 10 + [9920]
_STARTS = [0]
for _s in _SIZES:
    _STARTS.append(_STARTS[-1] + _s)
_HALF_E = _E // 2


def _edge_body(ea_ref, we1_ref, we2_ref, e_ref):
    e_ref[...] = jax.nn.silu(ea_ref[...] @ we1_ref[...]) @ we2_ref[...]


def _tc_edge(edge_attr, We1, We2):
    te = 2560
    return pl.pallas_call(
        _edge_body,
        grid=(_E // te,),
        in_specs=[
            pl.BlockSpec((te, _DE), lambda i: (i, 0)),
            pl.BlockSpec((_DE, _D), lambda i: (0, 0)),
            pl.BlockSpec((_D, _D), lambda i: (0, 0)),
        ],
        out_specs=pl.BlockSpec((te, _D), lambda i: (i, 0)),
        out_shape=jax.ShapeDtypeStruct((_E, _D), _F32),
    )(edge_attr, We1, We2)


def _h0_body(x_ref, wenc_ref, benc_ref, h_ref):
    h_ref[...] = x_ref[...] @ wenc_ref[...] + benc_ref[...]


def _tc_h0(x, Wenc, benc2):
    tn = 1000
    return pl.pallas_call(
        _h0_body,
        grid=(_N // tn,),
        in_specs=[
            pl.BlockSpec((tn, _DIN), lambda i: (i, 0)),
            pl.BlockSpec((_DIN, _D), lambda i: (0, 0)),
            pl.BlockSpec((1, _D), lambda i: (0, 0)),
        ],
        out_specs=pl.BlockSpec((tn, _D), lambda i: (i, 0)),
        out_shape=jax.ShapeDtypeStruct((_N, _D), _F32),
    )(x, Wenc, benc2)


def _msg_body(hg_ref, es_ref, wmsg_ref, m_ref):
    m_ref[...] = jax.nn.silu((hg_ref[...] + es_ref[...]) @ wmsg_ref[...])


def _tc_msg(hg, e_s, Wmsg_l):
    te = 2560
    return pl.pallas_call(
        _msg_body,
        grid=(_E // te,),
        in_specs=[
            pl.BlockSpec((te, _D), lambda i: (i, 0)),
            pl.BlockSpec((te, _D), lambda i: (i, 0)),
            pl.BlockSpec((_D, _D), lambda i: (0, 0)),
        ],
        out_specs=pl.BlockSpec((te, _D), lambda i: (i, 0)),
        out_shape=jax.ShapeDtypeStruct((_E, _D), _F32),
    )(hg, e_s, Wmsg_l)


_TN_UPD = 1000


def _upd_body(h_ref, agg_ref, pf_ref, plt_ref, fn_ref, ln_ref, wupd_ref,
              hn_ref):
    i = pl.program_id(0)
    rows = lax.broadcasted_iota(_I32, (_TN_UPD, 1), 0) + i * _TN_UPD
    fn = fn_ref[...]
    ln = ln_ref[...]
    agg = agg_ref[...]
    keep = jnp.ones((_TN_UPD, 1), _F32)
    for c in range(32):
        keep = keep * jnp.where(rows == fn[c * 8, 0], 0.0, 1.0)
        keep = keep * jnp.where(rows == ln[c * 8, 0], 0.0, 1.0)
    agg = agg * keep
    for c in range(32):
        agg = agg + jnp.where(rows == fn[c * 8, 0], 1.0, 0.0) * pf_ref[c * 8]
        agg = agg + jnp.where(rows == ln[c * 8, 0], 1.0, 0.0) * plt_ref[c * 8]
    hn_ref[...] = h_ref[...] + jax.nn.silu(agg @ wupd_ref[...])


def _tc_update(h, agg_raw, partsF, partsL, fnid, lnid, Wupd_l):
    return pl.pallas_call(
        _upd_body,
        grid=(_N // _TN_UPD,),
        in_specs=[
            pl.BlockSpec((_TN_UPD, _D), lambda i: (i, 0)),
            pl.BlockSpec((_TN_UPD, _D), lambda i: (i, 0)),
            pl.BlockSpec((256, _D), lambda i: (0, 0)),
            pl.BlockSpec((256, _D), lambda i: (0, 0)),
            pl.BlockSpec((256, 16), lambda i: (0, 0)),
            pl.BlockSpec((256, 16), lambda i: (0, 0)),
            pl.BlockSpec((_D, _D), lambda i: (0, 0)),
        ],
        out_specs=pl.BlockSpec((_TN_UPD, _D), lambda i: (i, 0)),
        out_shape=jax.ShapeDtypeStruct((_N, _D), _F32),
    )(h, agg_raw, partsF, partsL, fnid, lnid, Wupd_l)


def _pool_body(h_ref, gid_ref, w1_ref, b1_ref, w2_ref, b2_ref, w3_ref, b3_ref,
               out_ref):
    ids = gid_ref[...]
    iota = lax.broadcasted_iota(_I32, (_N, _B), 1)
    onehot = (ids == iota).astype(_F32)
    pooled = lax.dot_general(onehot, h_ref[...], (((0,), (0,)), ((), ())),
                             precision=lax.Precision.HIGHEST)
    counts = jnp.maximum(jnp.sum(onehot, axis=0), 1.0)
    pooled = pooled / counts[:, None]
    o = jax.nn.silu(pooled @ w1_ref[...] + b1_ref[...])
    o = jax.nn.silu(o @ w2_ref[...] + b2_ref[...])
    out_ref[...] = o @ w3_ref[...] + b3_ref[...]


def _tc_pool_head(h, gid2, W1, b1_2, W2, b2_2, W3, b3_2):
    return pl.pallas_call(
        _pool_body,
        out_shape=jax.ShapeDtypeStruct((_B, _OUT), _F32),
    )(h, gid2, W1, b1_2, W2, b2_2, W3, b3_2)


_GPT = _E // 32
_GCH = 200
_GFULL = _GPT // _GCH


def _sc_take(table, idx):
    mesh = plsc.VectorSubcoreMesh(core_axis_name="c", subcore_axis_name="s")

    @functools.partial(
        pl.kernel,
        mesh=mesh,
        out_type=jax.ShapeDtypeStruct((_E, _D), _F32),
        scratch_types=[
            pltpu.VMEM((_GCH,), _I32),
            pltpu.VMEM((_GCH, _D), _F32),
            pltpu.SemaphoreType.DMA,
        ],
    )
    def k(t_h, idx_h, out_h, gidx, buf, sem1):
        cc = lax.axis_index("c")
        ss = lax.axis_index("s")
        wid = cc * 16 + ss
        base = wid * _GPT

        def chunk(i, carry):
            e0 = base + i * _GCH
            pltpu.sync_copy(idx_h.at[pl.ds(e0, _GCH)], gidx)
            cp1 = pltpu.async_copy(t_h.at[gidx], buf, sem1)
            cp1.wait()
            pltpu.sync_copy(buf, out_h.at[pl.ds(e0, _GCH)])
            return carry

        lax.fori_loop(0, _GFULL, chunk, 0)

    return k(table, idx)


_PART = 80
_SB = 16


def _sc_scatter(m_s, dst_s, nextfn):
    mesh = plsc.VectorSubcoreMesh(core_axis_name="c", subcore_axis_name="s")

    @functools.partial(
        pl.kernel,
        mesh=mesh,
        out_type=[
            jax.ShapeDtypeStruct((_NPAD, _D), _F32),
            jax.ShapeDtypeStruct((256, _D), _F32),
            jax.ShapeDtypeStruct((256, _D), _F32),
            jax.ShapeDtypeStruct((256, 16), _I32),
            jax.ShapeDtypeStruct((256, 16), _I32),
        ],
        scratch_types=[
            pltpu.VMEM((_PART, _D), _F32),
            pltpu.VMEM((_PART + 16,), _I32),
            pltpu.VMEM((8, _D), _F32),
            pltpu.VMEM((_SB, _D), _F32),
            pltpu.VMEM((_SB,), _I32),
            pltpu.VMEM((8, 16), _I32),
            pltpu.VMEM((16,), _I32),
        ],
    )
    def k(m_h, dst_h, nfn_h, agg_h, pf_h, pl_h, fn_h, ln_h,
          mbuf, dbuf, acc, sbuf, sidx, idbuf, nfb):
        cc = lax.axis_index("c")
        ss = lax.axis_index("s")
        chunkid = cc * 16 + ss
        prow = chunkid * 8
        st = jnp.where(ss < 5, 10080 * ss,
                       jnp.where(ss < 15, 50400 + 9968 * (ss - 5), 150080))
        st = st + cc * _HALF_E
        nfull = jnp.where(ss < 5, 126, 124)
        has_tail = jnp.logical_and(ss >= 5, ss < 15)
        dummy = _N + chunkid
        lanes = lax.iota(_I32, 16)
        zv = jnp.zeros((16,), _F32)

        def stage_acc(node, kst):
            for j in range(_D // 16):
                sl = pl.ds(j * 16, 16)
                sbuf[kst, sl] = acc[0, sl]
            sidx[...] = jnp.where(lanes == kst, node, sidx[...])
            return kst + 1

        def stage_zero(node, kst):
            for j in range(_D // 16):
                sbuf[kst, pl.ds(j * 16, 16)] = zv
            sidx[...] = jnp.where(lanes == kst, node, sidx[...])
            return kst + 1

        def maybe_flush(kst):
            @pl.when(kst == _SB)
            def _():
                pltpu.sync_copy(sbuf, agg_h.at[sidx])
            return jnp.where(kst == _SB, 0, kst)

        def gap_zeros(lo, hi, kst):
            def gap(g, kk):
                return maybe_flush(stage_zero(g, kk))
            return lax.fori_loop(lo, hi, gap, kst)

        def write_partial(dst_ref, id_ref, node):
            pltpu.sync_copy(acc, dst_ref.at[pl.ds(prow, 8)])
            idbuf[0, pl.ds(0, 16)] = jnp.full((16,), node, _I32)
            pltpu.sync_copy(idbuf, id_ref.at[pl.ds(prow, 8)])

        def process_part(e0, cnt, carry):
            cur, first_done, kst = carry
            pltpu.sync_copy(m_h.at[pl.ds(e0, cnt)], mbuf.at[pl.ds(0, cnt)])
            pltpu.sync_copy(dst_h.at[pl.ds(e0, cnt)], dbuf.at[pl.ds(0, cnt)])

            def edge(i, carry2):
                cur, first_done, kst = carry2
                d = dbuf[pl.ds(i, 16)][0]
                is_new = d != cur
                do_flush = jnp.logical_and(is_new, cur >= 0)

                def flush_and_gap():
                    def to_partsF():
                        write_partial(pf_h, fn_h, cur)
                        return kst

                    def to_stage():
                        return maybe_flush(stage_acc(cur, kst))

                    k2 = lax.cond(first_done == 0, to_partsF, to_stage)
                    return gap_zeros(cur + 1, d, k2)

                kst = lax.cond(do_flush, flush_and_gap, lambda: kst)

                @pl.when(is_new)
                def _():
                    for j in range(_D // 16):
                        sl = pl.ds(j * 16, 16)
                        acc[0, sl] = mbuf[i, sl]

                @pl.when(jnp.logical_not(is_new))
                def _():
                    for j in range(_D // 16):
                        sl = pl.ds(j * 16, 16)
                        acc[0, sl] = acc[0, sl] + mbuf[i, sl]

                fd = jnp.where(do_flush, 1, first_done)
                cur = jnp.where(is_new, d, cur)
                return (cur, fd, kst)

            return lax.fori_loop(0, cnt, edge, (cur, first_done, kst))

        pltpu.sync_copy(nfn_h.at[chunkid], nfb)
        nextfn = nfb[...][0]

        sidx[...] = jnp.full((16,), dummy, _I32)
        k0 = jnp.int32(0)

        @pl.when(chunkid == 0)
        def _():
            pltpu.sync_copy(dst_h.at[pl.ds(0, 16)], nfb)

        first0 = jnp.where(chunkid == 0, nfb[...][0], 0)

        @pl.when(chunkid == 0)
        def _():
            pltpu.sync_copy(nfn_h.at[chunkid], nfb)

        k0 = gap_zeros(0, first0, k0)

        def part(p, carry):
            return process_part(st + p * _PART, _PART, carry)

        carry = (jnp.int32(-1), jnp.int32(0), k0)
        carry = lax.fori_loop(0, nfull, part, carry)
        cur, first_done, kst = lax.cond(
            has_tail,
            lambda: process_part(st + nfull * _PART, 48, carry),
            lambda: carry)

        @pl.when(first_done == 0)
        def _():
            write_partial(pf_h, fn_h, cur)
            for j in range(_D // 16):
                acc[0, pl.ds(j * 16, 16)] = zv
            write_partial(pl_h, ln_h, -1)

        @pl.when(first_done != 0)
        def _():
            write_partial(pl_h, ln_h, cur)

        kst = gap_zeros(cur + 1, nextfn, kst)

        sidx[...] = jnp.where(lanes >= kst, dummy, sidx[...])
        pltpu.sync_copy(sbuf, agg_h.at[sidx])

    return k(m_s, dst_s, nextfn)


def kernel(x, edge_index, edge_attr, graph_ids, We1, We2, Wenc, benc,
           Wmsg, Wupd, W1, b1, W2, b2, W3, b3):
    src = edge_index[0]
    dst = edge_index[1]
    perm = jnp.argsort(dst, stable=True).astype(_I32)
    src_s = src[perm]
    dst_s = dst[perm]
    chunk_starts = jnp.array(
        [_STARTS[i] for i in range(1, 16)] + [_HALF_E]
        + [_HALF_E + _STARTS[i] for i in range(1, 16)], _I32)
    nextfn = jnp.concatenate([dst_s[chunk_starts], jnp.array([_N], _I32)])
    nextfn = jnp.broadcast_to(nextfn[:, None], (32, 16)).astype(_I32)

    benc2 = benc.reshape(1, _D)
    gid2 = graph_ids.reshape(_N, 1)
    b1_2 = b1.reshape(1, _H)
    b2_2 = b2.reshape(1, _H)
    b3_2 = b3.reshape(1, _OUT)

    e = _tc_edge(edge_attr, We1, We2)
    h = _tc_h0(x, Wenc, benc2)
    e_s = _sc_take(e, perm)

    for l in range(_L):
        hg = _sc_take(h, src_s)
        m_s = _tc_msg(hg, e_s, Wmsg[l])
        agg_raw, partsF, partsL, fnid, lnid = _sc_scatter(m_s, dst_s, nextfn)
        h = _tc_update(h, agg_raw[:_N], partsF, partsL, fnid, lnid, Wupd[l])

    return _tc_pool_head(h, gid2, W1, b1_2, W2, b2_2, W3, b3_2)
